# Initial kernel scaffold; baseline (speedup 1.0000x reference)
import jax
import jax.numpy as jnp
from jax.experimental import pallas as pl


def kernel(x, edge_index, W1, b1, W2, b2, Wp1, bp1, Wp2, bp2):
    raise NotImplementedError("write your pallas kernel here")


# speedup vs baseline: 12.3314x; 12.3314x over previous
import functools

import jax
import jax.numpy as jnp
from jax import lax
from jax.experimental import pallas as pl
from jax.experimental.pallas import tpu as pltpu
from jax.experimental.pallas import tpu_sc as plsc

N = 10000
NPAD = 10240
E = 320000
D = 128

NC = 2
NS = 16
NW = NC * NS
EPW = E // NW
C = 80
NCHUNK = EPW // C
RPT = NPAD // NS
DEG_W = 128

@functools.cache
def _sc_mesh():
    return plsc.VectorSubcoreMesh(core_axis_name="c", subcore_axis_name="s",
                                  num_cores=NC, num_subcores=NS)


def _deg_body(dst_hbm, ones_hbm, zeros_hbm, out_hbm, acc_sh, idx_v, ones_v):
    core = lax.axis_index("c")
    tid = lax.axis_index("s")
    wid = tid * NC + core
    pltpu.sync_copy(zeros_hbm.at[pl.ds(tid * RPT, RPT)],
                    acc_sh.at[pl.ds(tid * RPT, RPT)])
    pltpu.sync_copy(ones_hbm, ones_v)
    plsc.subcore_barrier()

    def chunk(ci, carry):
        b = wid * EPW + ci * C
        pltpu.sync_copy(dst_hbm.at[pl.ds(b, C)], idx_v)
        pltpu.sync_copy(ones_v, acc_sh.at[idx_v], add=True)
        return carry

    lax.fori_loop(0, NCHUNK, chunk, 0)
    plsc.subcore_barrier()
    pltpu.sync_copy(acc_sh.at[pl.ds(tid * RPT, RPT)],
                    out_hbm.at[core].at[pl.ds(tid * RPT, RPT)])


def _degree_partials(dst, ones_rows, zeros_rows):
    return pl.kernel(
        _deg_body,
        out_type=jax.ShapeDtypeStruct((NC, NPAD, DEG_W), jnp.float32),
        mesh=_sc_mesh(),
        scratch_types=[
            pltpu.VMEM_SHARED((NPAD, DEG_W), jnp.float32),
            pltpu.VMEM((C,), jnp.int32),
            pltpu.VMEM((C, DEG_W), jnp.float32),
        ],
    )(dst, ones_rows, zeros_rows)


def _gather_body(src_hbm, dst_hbm, g_hbm, zeros_hbm, out_hbm,
                 acc_sh, sidx, didx, rows, sem):
    core = lax.axis_index("c")
    tid = lax.axis_index("s")
    wid = tid * NC + core
    pltpu.sync_copy(zeros_hbm.at[pl.ds(tid * RPT, RPT)],
                    acc_sh.at[pl.ds(tid * RPT, RPT)])
    plsc.subcore_barrier()

    def chunk(ci, carry):
        b = wid * EPW + ci * C
        pltpu.sync_copy(src_hbm.at[pl.ds(b, C)], sidx)
        pltpu.sync_copy(dst_hbm.at[pl.ds(b, C)], didx)
        pltpu.async_copy(g_hbm.at[sidx], rows, sem).wait()
        pltpu.sync_copy(rows, acc_sh.at[didx], add=True)
        return carry

    lax.fori_loop(0, NCHUNK, chunk, 0)
    plsc.subcore_barrier()
    pltpu.sync_copy(acc_sh.at[pl.ds(tid * RPT, RPT)],
                    out_hbm.at[core].at[pl.ds(tid * RPT, RPT)])


def _message_partials(src, dst, g, zeros_rows):
    return pl.kernel(
        _gather_body,
        out_type=jax.ShapeDtypeStruct((NC, NPAD, D), jnp.float32),
        mesh=_sc_mesh(),
        scratch_types=[
            pltpu.VMEM_SHARED((NPAD, D), jnp.float32),
            pltpu.VMEM((C,), jnp.int32),
            pltpu.VMEM((C,), jnp.int32),
            pltpu.VMEM((C, D), jnp.float32),
            pltpu.SemaphoreType.DMA,
        ],
    )(src, dst, g, zeros_rows)


R = 2000


def _dis_from(dp_ref):
    deg = dp_ref[0, :, 0] + dp_ref[1, :, 0] + 1.0
    return lax.rsqrt(deg)


def _mm(a, w):
    return lax.dot_general(a, w, (((1,), (1,)), ((), ())),
                           preferred_element_type=jnp.float32)


def _tc_first_body(dp_ref, x_ref, w1_ref, g1_ref):
    dis = _dis_from(dp_ref)
    g1_ref[...] = _mm(x_ref[...], w1_ref[...]) * dis[:, None]


def _tc_mid_body(dp_ref, acc_ref, g1_ref, b1_ref, w2_ref, x1_ref, g2_ref):
    dis = _dis_from(dp_ref)
    t = (acc_ref[0] + acc_ref[1] + g1_ref[...]) * dis[:, None] + b1_ref[...]
    x1 = jnp.maximum(t, 0.0)
    x1_ref[...] = x1
    g2_ref[...] = _mm(x1, w2_ref[...]) * dis[:, None]


def _tc_last_body(dp_ref, acc_ref, g2_ref, b2_ref, x1_ref,
                  wp1a_ref, wp1b_ref, bp1_ref, wp2_ref, bp2_ref, out_ref):
    dis = _dis_from(dp_ref)
    t = (acc_ref[0] + acc_ref[1] + g2_ref[...]) * dis[:, None] + b2_ref[...]
    x2 = jnp.maximum(t, 0.0)
    p = _mm(x1_ref[...], wp1a_ref[...]) + _mm(x2, wp1b_ref[...]) + bp1_ref[...]
    p = jnp.maximum(p, 0.0)
    out_ref[...] = _mm(p, wp2_ref[...]) + bp2_ref[...]


def _row_spec(width):
    return pl.BlockSpec((R, width), lambda i: (i, 0))


_DP_SPEC = pl.BlockSpec((NC, R, DEG_W), lambda i: (0, i, 0))
_ACC_SPEC = pl.BlockSpec((NC, R, D), lambda i: (0, i, 0))


def _full_spec(r, c):
    return pl.BlockSpec((r, c), lambda i: (0, 0))


def _tc_first(dp, x, w1):
    return pl.pallas_call(
        _tc_first_body,
        grid=(N // R,),
        in_specs=[_DP_SPEC, _row_spec(D), _full_spec(D, D)],
        out_specs=_row_spec(D),
        out_shape=jax.ShapeDtypeStruct((N, D), jnp.float32),
    )(dp, x, w1)


def _tc_mid(dp, acc, g1, b1, w2):
    return pl.pallas_call(
        _tc_mid_body,
        grid=(N // R,),
        in_specs=[_DP_SPEC, _ACC_SPEC, _row_spec(D), _full_spec(1, D),
                  _full_spec(D, D)],
        out_specs=[_row_spec(D), _row_spec(D)],
        out_shape=[jax.ShapeDtypeStruct((N, D), jnp.float32),
                   jax.ShapeDtypeStruct((N, D), jnp.float32)],
    )(dp, acc, g1, b1, w2)


def _tc_last(dp, acc, g2, b2, x1, wp1a, wp1b, bp1, wp2, bp2):
    return pl.pallas_call(
        _tc_last_body,
        grid=(N // R,),
        in_specs=[_DP_SPEC, _ACC_SPEC, _row_spec(D), _full_spec(1, D),
                  _row_spec(D), _full_spec(D, D), _full_spec(D, D),
                  _full_spec(1, D), _full_spec(D, D), _full_spec(1, D)],
        out_specs=_row_spec(D),
        out_shape=jax.ShapeDtypeStruct((N, D), jnp.float32),
    )(dp, acc, g2, b2, x1, wp1a, wp1b, bp1, wp2, bp2)


def kernel(x, edge_index, W1, b1, W2, b2, Wp1, bp1, Wp2, bp2):
    src = edge_index[0]
    dst = edge_index[1]
    ones_rows = jnp.ones((C, DEG_W), jnp.float32)
    zeros_g = jnp.zeros((NPAD, D), jnp.float32)

    dp = _degree_partials(dst, ones_rows, zeros_g)

    g1 = _tc_first(dp, x, W1)
    acc1 = _message_partials(src, dst, g1, zeros_g)
    x1, g2 = _tc_mid(dp, acc1, g1, b1.reshape(1, D), W2)
    acc2 = _message_partials(src, dst, g2, zeros_g)
    out = _tc_last(dp, acc2, g2, b2.reshape(1, D), x1,
                   Wp1[:, :D], Wp1[:, D:], bp1.reshape(1, D), Wp2,
                   bp2.reshape(1, D))
    return out


# speedup vs baseline: 23.7064x; 1.9224x over previous
import functools

import jax
import jax.numpy as jnp
from jax import lax
from jax.experimental import pallas as pl
from jax.experimental.pallas import tpu as pltpu
from jax.experimental.pallas import tpu_sc as plsc

N = 10000
NPAD = 10240
E = 320000
D = 128

NC = 2
NS = 16
NW = NC * NS
EPW = E // NW
C = 40
NCHUNK = EPW // C
RPT = NPAD // NS
DEG_W = 128

@functools.cache
def _sc_mesh():
    return plsc.VectorSubcoreMesh(core_axis_name="c", subcore_axis_name="s",
                                  num_cores=NC, num_subcores=NS)


NBUF = 5
NGRP = NCHUNK // NBUF


def _deg_body(dst_hbm, ones_hbm, zeros_hbm, out_hbm, acc_sh,
              didx, ones_v, sem_d, sem_s):
    core = lax.axis_index("c")
    tid = lax.axis_index("s")
    wid = tid * NC + core
    base = wid * EPW
    pltpu.sync_copy(zeros_hbm.at[pl.ds(tid * RPT, RPT)],
                    acc_sh.at[pl.ds(tid * RPT, RPT)])
    pltpu.sync_copy(ones_hbm, ones_v)
    plsc.subcore_barrier()

    def group(g, carry):
        for b in range(NBUF):
            @pl.when(g > 0)
            def _():
                pltpu.make_async_copy(ones_v, acc_sh.at[didx.at[b]],
                                      sem_s.at[b]).wait()
            pltpu.async_copy(dst_hbm.at[pl.ds(base + (g * NBUF + b) * C, C)],
                             didx.at[b], sem_d.at[b])
        for b in range(NBUF):
            pltpu.make_async_copy(dst_hbm.at[pl.ds(0, C)], didx.at[b],
                                  sem_d.at[b]).wait()
            pltpu.async_copy(ones_v, acc_sh.at[didx.at[b]], sem_s.at[b],
                             add=True)
        return carry

    lax.fori_loop(0, NGRP, group, 0)
    for b in range(NBUF):
        pltpu.make_async_copy(ones_v, acc_sh.at[didx.at[b]], sem_s.at[b]).wait()
    plsc.subcore_barrier()
    pltpu.sync_copy(acc_sh.at[pl.ds(tid * RPT, RPT)],
                    out_hbm.at[core].at[pl.ds(tid * RPT, RPT)])


def _degree_partials(dst, ones_rows, zeros_rows):
    return pl.kernel(
        _deg_body,
        out_type=jax.ShapeDtypeStruct((NC, NPAD, DEG_W), jnp.float32),
        mesh=_sc_mesh(),
        scratch_types=[
            pltpu.VMEM_SHARED((NPAD, DEG_W), jnp.float32),
            pltpu.VMEM((NBUF, C), jnp.int32),
            pltpu.VMEM((C, DEG_W), jnp.float32),
            pltpu.SemaphoreType.DMA((NBUF,)),
            pltpu.SemaphoreType.DMA((NBUF,)),
        ],
    )(dst, ones_rows, zeros_rows)


def _gather_body(src_hbm, dst_hbm, g_hbm, zeros_hbm, out_hbm,
                 acc_sh, sidx, didx, rows, sem_i, sem_d, sem_g, sem_s):
    core = lax.axis_index("c")
    tid = lax.axis_index("s")
    wid = tid * NC + core
    base = wid * EPW
    pltpu.sync_copy(zeros_hbm.at[pl.ds(tid * RPT, RPT)],
                    acc_sh.at[pl.ds(tid * RPT, RPT)])
    plsc.subcore_barrier()

    def group(g, carry):
        for b in range(NBUF):
            @pl.when(g > 0)
            def _():
                pltpu.make_async_copy(rows.at[b], acc_sh.at[didx.at[b]],
                                      sem_s.at[b]).wait()
            e = base + (g * NBUF + b) * C
            pltpu.async_copy(src_hbm.at[pl.ds(e, C)], sidx.at[b], sem_i.at[b])
            pltpu.async_copy(dst_hbm.at[pl.ds(e, C)], didx.at[b], sem_d.at[b])
        for b in range(NBUF):
            pltpu.make_async_copy(src_hbm.at[pl.ds(0, C)], sidx.at[b],
                                  sem_i.at[b]).wait()
            pltpu.async_copy(g_hbm.at[sidx.at[b]], rows.at[b], sem_g.at[b])
        for b in range(NBUF):
            pltpu.make_async_copy(g_hbm.at[sidx.at[b]], rows.at[b],
                                  sem_g.at[b]).wait()
            pltpu.make_async_copy(dst_hbm.at[pl.ds(0, C)], didx.at[b],
                                  sem_d.at[b]).wait()
            pltpu.async_copy(rows.at[b], acc_sh.at[didx.at[b]], sem_s.at[b],
                             add=True)
        return carry

    lax.fori_loop(0, NGRP, group, 0)
    for b in range(NBUF):
        pltpu.make_async_copy(rows.at[b], acc_sh.at[didx.at[b]],
                              sem_s.at[b]).wait()
    plsc.subcore_barrier()
    pltpu.sync_copy(acc_sh.at[pl.ds(tid * RPT, RPT)],
                    out_hbm.at[core].at[pl.ds(tid * RPT, RPT)])


def _message_partials(src, dst, g, zeros_rows):
    return pl.kernel(
        _gather_body,
        out_type=jax.ShapeDtypeStruct((NC, NPAD, D), jnp.float32),
        mesh=_sc_mesh(),
        scratch_types=[
            pltpu.VMEM_SHARED((NPAD, D), jnp.float32),
            pltpu.VMEM((NBUF, C), jnp.int32),
            pltpu.VMEM((NBUF, C), jnp.int32),
            pltpu.VMEM((NBUF, C, D), jnp.float32),
            pltpu.SemaphoreType.DMA((NBUF,)),
            pltpu.SemaphoreType.DMA((NBUF,)),
            pltpu.SemaphoreType.DMA((NBUF,)),
            pltpu.SemaphoreType.DMA((NBUF,)),
        ],
    )(src, dst, g, zeros_rows)


R = 2000


def _dis_from(dp_ref):
    deg = dp_ref[0, :, 0] + dp_ref[1, :, 0] + 1.0
    return lax.rsqrt(deg)


def _mm(a, w):
    return lax.dot_general(a, w, (((1,), (1,)), ((), ())),
                           preferred_element_type=jnp.float32)


def _tc_first_body(dp_ref, x_ref, w1_ref, g1_ref):
    dis = _dis_from(dp_ref)
    g1_ref[...] = _mm(x_ref[...], w1_ref[...]) * dis[:, None]


def _tc_mid_body(dp_ref, acc_ref, g1_ref, b1_ref, w2_ref, x1_ref, g2_ref):
    dis = _dis_from(dp_ref)
    t = (acc_ref[0] + acc_ref[1] + g1_ref[...]) * dis[:, None] + b1_ref[...]
    x1 = jnp.maximum(t, 0.0)
    x1_ref[...] = x1
    g2_ref[...] = _mm(x1, w2_ref[...]) * dis[:, None]


def _tc_last_body(dp_ref, acc_ref, g2_ref, b2_ref, x1_ref,
                  wp1a_ref, wp1b_ref, bp1_ref, wp2_ref, bp2_ref, out_ref):
    dis = _dis_from(dp_ref)
    t = (acc_ref[0] + acc_ref[1] + g2_ref[...]) * dis[:, None] + b2_ref[...]
    x2 = jnp.maximum(t, 0.0)
    p = _mm(x1_ref[...], wp1a_ref[...]) + _mm(x2, wp1b_ref[...]) + bp1_ref[...]
    p = jnp.maximum(p, 0.0)
    out_ref[...] = _mm(p, wp2_ref[...]) + bp2_ref[...]


def _row_spec(width):
    return pl.BlockSpec((R, width), lambda i: (i, 0))


_DP_SPEC = pl.BlockSpec((NC, R, DEG_W), lambda i: (0, i, 0))
_ACC_SPEC = pl.BlockSpec((NC, R, D), lambda i: (0, i, 0))


def _full_spec(r, c):
    return pl.BlockSpec((r, c), lambda i: (0, 0))


def _tc_first(dp, x, w1):
    return pl.pallas_call(
        _tc_first_body,
        grid=(N // R,),
        in_specs=[_DP_SPEC, _row_spec(D), _full_spec(D, D)],
        out_specs=_row_spec(D),
        out_shape=jax.ShapeDtypeStruct((N, D), jnp.float32),
    )(dp, x, w1)


def _tc_mid(dp, acc, g1, b1, w2):
    return pl.pallas_call(
        _tc_mid_body,
        grid=(N // R,),
        in_specs=[_DP_SPEC, _ACC_SPEC, _row_spec(D), _full_spec(1, D),
                  _full_spec(D, D)],
        out_specs=[_row_spec(D), _row_spec(D)],
        out_shape=[jax.ShapeDtypeStruct((N, D), jnp.float32),
                   jax.ShapeDtypeStruct((N, D), jnp.float32)],
    )(dp, acc, g1, b1, w2)


def _tc_last(dp, acc, g2, b2, x1, wp1a, wp1b, bp1, wp2, bp2):
    return pl.pallas_call(
        _tc_last_body,
        grid=(N // R,),
        in_specs=[_DP_SPEC, _ACC_SPEC, _row_spec(D), _full_spec(1, D),
                  _row_spec(D), _full_spec(D, D), _full_spec(D, D),
                  _full_spec(1, D), _full_spec(D, D), _full_spec(1, D)],
        out_specs=_row_spec(D),
        out_shape=jax.ShapeDtypeStruct((N, D), jnp.float32),
    )(dp, acc, g2, b2, x1, wp1a, wp1b, bp1, wp2, bp2)


def kernel(x, edge_index, W1, b1, W2, b2, Wp1, bp1, Wp2, bp2):
    src = edge_index[0]
    dst = edge_index[1]
    ones_rows = jnp.ones((C, DEG_W), jnp.float32)
    zeros_g = jnp.zeros((NPAD, D), jnp.float32)

    dp = _degree_partials(dst, ones_rows, zeros_g)

    g1 = _tc_first(dp, x, W1)
    acc1 = _message_partials(src, dst, g1, zeros_g)
    x1, g2 = _tc_mid(dp, acc1, g1, b1.reshape(1, D), W2)
    acc2 = _message_partials(src, dst, g2, zeros_g)
    out = _tc_last(dp, acc2, g2, b2.reshape(1, D), x1,
                   Wp1[:, :D], Wp1[:, D:], bp1.reshape(1, D), Wp2,
                   bp2.reshape(1, D))
    return out


# speedup vs baseline: 24.3378x; 1.0266x over previous
import functools

import jax
import jax.numpy as jnp
from jax import lax
from jax.experimental import pallas as pl
from jax.experimental.pallas import tpu as pltpu
from jax.experimental.pallas import tpu_sc as plsc

N = 10000
NPAD = 10240
E = 320000
D = 128

NC = 2
NS = 16
NW = NC * NS
EPW = E // NW
C = 40
NCHUNK = EPW // C
RPT = NPAD // NS
DEG_W = 128

@functools.cache
def _sc_mesh():
    return plsc.VectorSubcoreMesh(core_axis_name="c", subcore_axis_name="s",
                                  num_cores=NC, num_subcores=NS)


NBUF = 5
NGRP = NCHUNK // NBUF


def _deg_body(dst_hbm, ones_hbm, zeros_hbm, out_hbm, acc_sh,
              didx, ones_v, sem_d, sem_s):
    core = lax.axis_index("c")
    tid = lax.axis_index("s")
    wid = tid * NC + core
    base = wid * EPW
    pltpu.sync_copy(zeros_hbm.at[pl.ds(tid * RPT, RPT)],
                    acc_sh.at[pl.ds(tid * RPT, RPT)])
    pltpu.sync_copy(ones_hbm, ones_v)
    plsc.subcore_barrier()

    def group(g, carry):
        for b in range(NBUF):
            @pl.when(g > 0)
            def _():
                pltpu.make_async_copy(ones_v, acc_sh.at[didx.at[b]],
                                      sem_s.at[b]).wait()
            pltpu.async_copy(dst_hbm.at[pl.ds(base + (g * NBUF + b) * C, C)],
                             didx.at[b], sem_d.at[b])
        for b in range(NBUF):
            pltpu.make_async_copy(dst_hbm.at[pl.ds(0, C)], didx.at[b],
                                  sem_d.at[b]).wait()
            pltpu.async_copy(ones_v, acc_sh.at[didx.at[b]], sem_s.at[b],
                             add=True)
        return carry

    lax.fori_loop(0, NGRP, group, 0)
    for b in range(NBUF):
        pltpu.make_async_copy(ones_v, acc_sh.at[didx.at[b]], sem_s.at[b]).wait()
    plsc.subcore_barrier()
    pltpu.sync_copy(acc_sh.at[pl.ds(tid * RPT, RPT)],
                    out_hbm.at[core].at[pl.ds(tid * RPT, RPT)])


def _degree_partials(dst, ones_rows, zeros_rows):
    return pl.kernel(
        _deg_body,
        out_type=jax.ShapeDtypeStruct((NC, NPAD, DEG_W), jnp.float32),
        mesh=_sc_mesh(),
        scratch_types=[
            pltpu.VMEM_SHARED((NPAD, DEG_W), jnp.float32),
            pltpu.VMEM((NBUF, C), jnp.int32),
            pltpu.VMEM((C, DEG_W), jnp.float32),
            pltpu.SemaphoreType.DMA((NBUF,)),
            pltpu.SemaphoreType.DMA((NBUF,)),
        ],
    )(dst, ones_rows, zeros_rows)


CM = 80
NB_M = 4
NCHUNK_M = EPW // CM
NG_M = NCHUNK_M // NB_M


def _gather_body(src_hbm, dst_hbm, g_hbm, zeros_hbm, out_hbm,
                 acc_sh, sidx, didx, rows, sem_i, sem_d, sem_g, sem_s):
    core = lax.axis_index("c")
    tid = lax.axis_index("s")
    wid = tid * NC + core
    base = wid * EPW
    pltpu.sync_copy(zeros_hbm.at[pl.ds(tid * RPT, RPT)],
                    acc_sh.at[pl.ds(tid * RPT, RPT)])
    plsc.subcore_barrier()

    def group(g, carry):
        for b in range(NB_M):
            @pl.when(g > 0)
            def _():
                pltpu.make_async_copy(rows.at[b], acc_sh.at[didx.at[b]],
                                      sem_s.at[b]).wait()
            e = base + (g * NB_M + b) * CM
            pltpu.async_copy(src_hbm.at[pl.ds(e, CM)], sidx.at[b], sem_i.at[b])
            pltpu.async_copy(dst_hbm.at[pl.ds(e, CM)], didx.at[b], sem_d.at[b])
        for b in range(NB_M):
            pltpu.make_async_copy(src_hbm.at[pl.ds(0, CM)], sidx.at[b],
                                  sem_i.at[b]).wait()
            pltpu.async_copy(g_hbm.at[sidx.at[b]], rows.at[b], sem_g.at[b])
        for b in range(NB_M):
            pltpu.make_async_copy(g_hbm.at[sidx.at[b]], rows.at[b],
                                  sem_g.at[b]).wait()
            pltpu.make_async_copy(dst_hbm.at[pl.ds(0, CM)], didx.at[b],
                                  sem_d.at[b]).wait()
            pltpu.async_copy(rows.at[b], acc_sh.at[didx.at[b]], sem_s.at[b],
                             add=True)
        return carry

    lax.fori_loop(0, NG_M, group, 0)
    pltpu.make_async_copy(rows.at[0], acc_sh.at[didx.at[0]], sem_s.at[0]).wait()
    e = base + NG_M * NB_M * CM
    pltpu.async_copy(src_hbm.at[pl.ds(e, CM)], sidx.at[0], sem_i.at[0])
    pltpu.async_copy(dst_hbm.at[pl.ds(e, CM)], didx.at[0], sem_d.at[0])
    pltpu.make_async_copy(src_hbm.at[pl.ds(0, CM)], sidx.at[0], sem_i.at[0]).wait()
    pltpu.async_copy(g_hbm.at[sidx.at[0]], rows.at[0], sem_g.at[0])
    pltpu.make_async_copy(g_hbm.at[sidx.at[0]], rows.at[0], sem_g.at[0]).wait()
    pltpu.make_async_copy(dst_hbm.at[pl.ds(0, CM)], didx.at[0], sem_d.at[0]).wait()
    pltpu.async_copy(rows.at[0], acc_sh.at[didx.at[0]], sem_s.at[0], add=True)
    pltpu.make_async_copy(rows.at[0], acc_sh.at[didx.at[0]], sem_s.at[0]).wait()
    for b in range(1, NB_M):
        pltpu.make_async_copy(rows.at[b], acc_sh.at[didx.at[b]],
                              sem_s.at[b]).wait()
    plsc.subcore_barrier()
    pltpu.sync_copy(acc_sh.at[pl.ds(tid * RPT, RPT)],
                    out_hbm.at[core].at[pl.ds(tid * RPT, RPT)])


def _message_partials(src, dst, g, zeros_rows):
    return pl.kernel(
        _gather_body,
        out_type=jax.ShapeDtypeStruct((NC, NPAD, D), jnp.float32),
        mesh=_sc_mesh(),
        scratch_types=[
            pltpu.VMEM_SHARED((NPAD, D), jnp.float32),
            pltpu.VMEM((NB_M, CM), jnp.int32),
            pltpu.VMEM((NB_M, CM), jnp.int32),
            pltpu.VMEM((NB_M, CM, D), jnp.float32),
            pltpu.SemaphoreType.DMA((NB_M,)),
            pltpu.SemaphoreType.DMA((NB_M,)),
            pltpu.SemaphoreType.DMA((NB_M,)),
            pltpu.SemaphoreType.DMA((NB_M,)),
        ],
    )(src, dst, g, zeros_rows)


R = 2000


def _dis_from(dp_ref):
    deg = dp_ref[0, :, 0] + dp_ref[1, :, 0] + 1.0
    return lax.rsqrt(deg)


def _mm(a, w):
    return lax.dot_general(a, w, (((1,), (1,)), ((), ())),
                           preferred_element_type=jnp.float32)


def _tc_first_body(dp_ref, x_ref, w1_ref, g1_ref):
    dis = _dis_from(dp_ref)
    g1_ref[...] = _mm(x_ref[...], w1_ref[...]) * dis[:, None]


def _tc_mid_body(dp_ref, acc_ref, g1_ref, b1_ref, w2_ref, x1_ref, g2_ref):
    dis = _dis_from(dp_ref)
    t = (acc_ref[0] + acc_ref[1] + g1_ref[...]) * dis[:, None] + b1_ref[...]
    x1 = jnp.maximum(t, 0.0)
    x1_ref[...] = x1
    g2_ref[...] = _mm(x1, w2_ref[...]) * dis[:, None]


def _tc_last_body(dp_ref, acc_ref, g2_ref, b2_ref, x1_ref,
                  wp1a_ref, wp1b_ref, bp1_ref, wp2_ref, bp2_ref, out_ref):
    dis = _dis_from(dp_ref)
    t = (acc_ref[0] + acc_ref[1] + g2_ref[...]) * dis[:, None] + b2_ref[...]
    x2 = jnp.maximum(t, 0.0)
    p = _mm(x1_ref[...], wp1a_ref[...]) + _mm(x2, wp1b_ref[...]) + bp1_ref[...]
    p = jnp.maximum(p, 0.0)
    out_ref[...] = _mm(p, wp2_ref[...]) + bp2_ref[...]


def _row_spec(width):
    return pl.BlockSpec((R, width), lambda i: (i, 0))


_DP_SPEC = pl.BlockSpec((NC, R, DEG_W), lambda i: (0, i, 0))
_ACC_SPEC = pl.BlockSpec((NC, R, D), lambda i: (0, i, 0))


def _full_spec(r, c):
    return pl.BlockSpec((r, c), lambda i: (0, 0))


def _tc_first(dp, x, w1):
    return pl.pallas_call(
        _tc_first_body,
        grid=(N // R,),
        in_specs=[_DP_SPEC, _row_spec(D), _full_spec(D, D)],
        out_specs=_row_spec(D),
        out_shape=jax.ShapeDtypeStruct((N, D), jnp.float32),
    )(dp, x, w1)


def _tc_mid(dp, acc, g1, b1, w2):
    return pl.pallas_call(
        _tc_mid_body,
        grid=(N // R,),
        in_specs=[_DP_SPEC, _ACC_SPEC, _row_spec(D), _full_spec(1, D),
                  _full_spec(D, D)],
        out_specs=[_row_spec(D), _row_spec(D)],
        out_shape=[jax.ShapeDtypeStruct((N, D), jnp.float32),
                   jax.ShapeDtypeStruct((N, D), jnp.float32)],
    )(dp, acc, g1, b1, w2)


def _tc_last(dp, acc, g2, b2, x1, wp1a, wp1b, bp1, wp2, bp2):
    return pl.pallas_call(
        _tc_last_body,
        grid=(N // R,),
        in_specs=[_DP_SPEC, _ACC_SPEC, _row_spec(D), _full_spec(1, D),
                  _row_spec(D), _full_spec(D, D), _full_spec(D, D),
                  _full_spec(1, D), _full_spec(D, D), _full_spec(1, D)],
        out_specs=_row_spec(D),
        out_shape=jax.ShapeDtypeStruct((N, D), jnp.float32),
    )(dp, acc, g2, b2, x1, wp1a, wp1b, bp1, wp2, bp2)


def kernel(x, edge_index, W1, b1, W2, b2, Wp1, bp1, Wp2, bp2):
    src = edge_index[0]
    dst = edge_index[1]
    ones_rows = jnp.ones((C, DEG_W), jnp.float32)
    zeros_g = jnp.zeros((NPAD, D), jnp.float32)

    dp = _degree_partials(dst, ones_rows, zeros_g)

    g1 = _tc_first(dp, x, W1)
    acc1 = _message_partials(src, dst, g1, zeros_g)
    x1, g2 = _tc_mid(dp, acc1, g1, b1.reshape(1, D), W2)
    acc2 = _message_partials(src, dst, g2, zeros_g)
    out = _tc_last(dp, acc2, g2, b2.reshape(1, D), x1,
                   Wp1[:, :D], Wp1[:, D:], bp1.reshape(1, D), Wp2,
                   bp2.reshape(1, D))
    return out


# speedup vs baseline: 25.8978x; 1.0641x over previous
import functools

import jax
import jax.numpy as jnp
from jax import lax
from jax.experimental import pallas as pl
from jax.experimental.pallas import tpu as pltpu
from jax.experimental.pallas import tpu_sc as plsc

N = 10000
NPAD = 10240
E = 320000
D = 128

NC = 2
NS = 16
NW = NC * NS
EPW = E // NW
C = 40
NCHUNK = EPW // C
RPT = NPAD // NS
DEG_W = 128

@functools.cache
def _sc_mesh():
    return plsc.VectorSubcoreMesh(core_axis_name="c", subcore_axis_name="s",
                                  num_cores=NC, num_subcores=NS)


DGC = 80
DGB = 4
NG_D = (EPW // DGC) // DGB


def _deg_body(eflat_hbm, out_hbm, acc_sh, didx, ones_v, zbuf, sem_d, sem_s):
    core = lax.axis_index("c")
    tid = lax.axis_index("s")
    wid = tid * NC + core
    base = E + wid * EPW
    ones16 = jnp.ones((16,), jnp.float32)
    zeros16 = jnp.zeros((16,), jnp.float32)

    def fill(r, carry):
        for l in range(DEG_W // 16):
            ones_v[r, pl.ds(l * 16, 16)] = ones16
            zbuf[r, pl.ds(l * 16, 16)] = zeros16
        return carry

    lax.fori_loop(0, DGC, fill, 0)
    for q in range(RPT // DGC):
        pltpu.sync_copy(zbuf, acc_sh.at[pl.ds(tid * RPT + q * DGC, DGC)])
    plsc.subcore_barrier()

    def group(g, carry):
        for b in range(DGB):
            @pl.when(g > 0)
            def _():
                pltpu.make_async_copy(ones_v, acc_sh.at[didx.at[b]],
                                      sem_s.at[b]).wait()
            pltpu.async_copy(eflat_hbm.at[pl.ds(base + (g * DGB + b) * DGC, DGC)],
                             didx.at[b], sem_d.at[b])
        for b in range(DGB):
            pltpu.make_async_copy(eflat_hbm.at[pl.ds(0, DGC)], didx.at[b],
                                  sem_d.at[b]).wait()
            pltpu.async_copy(ones_v, acc_sh.at[didx.at[b]], sem_s.at[b],
                             add=True)
        return carry

    lax.fori_loop(0, NG_D, group, 0)
    pltpu.make_async_copy(ones_v, acc_sh.at[didx.at[0]], sem_s.at[0]).wait()
    pltpu.async_copy(eflat_hbm.at[pl.ds(base + NG_D * DGB * DGC, DGC)],
                     didx.at[0], sem_d.at[0])
    pltpu.make_async_copy(eflat_hbm.at[pl.ds(0, DGC)], didx.at[0],
                          sem_d.at[0]).wait()
    pltpu.async_copy(ones_v, acc_sh.at[didx.at[0]], sem_s.at[0], add=True)
    for b in range(DGB):
        pltpu.make_async_copy(ones_v, acc_sh.at[didx.at[b]], sem_s.at[b]).wait()
    plsc.subcore_barrier()
    pltpu.sync_copy(acc_sh.at[pl.ds(tid * RPT, RPT)],
                    out_hbm.at[core].at[pl.ds(tid * RPT, RPT)])


def _degree_partials(eflat):
    return pl.kernel(
        _deg_body,
        out_type=jax.ShapeDtypeStruct((NC, NPAD, DEG_W), jnp.float32),
        mesh=_sc_mesh(),
        scratch_types=[
            pltpu.VMEM_SHARED((NPAD, DEG_W), jnp.float32),
            pltpu.VMEM((DGB, DGC), jnp.int32),
            pltpu.VMEM((DGC, DEG_W), jnp.float32),
            pltpu.VMEM((DGC, DEG_W), jnp.float32),
            pltpu.SemaphoreType.DMA((DGB,)),
            pltpu.SemaphoreType.DMA((DGB,)),
        ],
    )(eflat)


CM = 80
NB_M = 4
NCHUNK_M = EPW // CM
NG_M = NCHUNK_M // NB_M


def _gather_body(eflat_hbm, g_hbm, out_hbm,
                 acc_sh, sidx, didx, rows, sem_i, sem_d, sem_g, sem_s):
    core = lax.axis_index("c")
    tid = lax.axis_index("s")
    wid = tid * NC + core
    base = wid * EPW
    zeros16 = jnp.zeros((16,), jnp.float32)

    def fill(r, carry):
        for l in range(D // 16):
            rows[0, r, pl.ds(l * 16, 16)] = zeros16
        return carry

    lax.fori_loop(0, CM, fill, 0)
    for q in range(RPT // CM):
        pltpu.sync_copy(rows.at[0], acc_sh.at[pl.ds(tid * RPT + q * CM, CM)])
    plsc.subcore_barrier()

    def group(g, carry):
        for b in range(NB_M):
            @pl.when(g > 0)
            def _():
                pltpu.make_async_copy(rows.at[b], acc_sh.at[didx.at[b]],
                                      sem_s.at[b]).wait()
            e = base + (g * NB_M + b) * CM
            pltpu.async_copy(eflat_hbm.at[pl.ds(e, CM)], sidx.at[b], sem_i.at[b])
            pltpu.async_copy(eflat_hbm.at[pl.ds(E + e, CM)], didx.at[b], sem_d.at[b])
        for b in range(NB_M):
            pltpu.make_async_copy(eflat_hbm.at[pl.ds(0, CM)], sidx.at[b],
                                  sem_i.at[b]).wait()
            pltpu.async_copy(g_hbm.at[sidx.at[b]], rows.at[b], sem_g.at[b])
        for b in range(NB_M):
            pltpu.make_async_copy(g_hbm.at[sidx.at[b]], rows.at[b],
                                  sem_g.at[b]).wait()
            pltpu.make_async_copy(eflat_hbm.at[pl.ds(0, CM)], didx.at[b],
                                  sem_d.at[b]).wait()
            pltpu.async_copy(rows.at[b], acc_sh.at[didx.at[b]], sem_s.at[b],
                             add=True)
        return carry

    lax.fori_loop(0, NG_M, group, 0)
    pltpu.make_async_copy(rows.at[0], acc_sh.at[didx.at[0]], sem_s.at[0]).wait()
    e = base + NG_M * NB_M * CM
    pltpu.async_copy(eflat_hbm.at[pl.ds(e, CM)], sidx.at[0], sem_i.at[0])
    pltpu.async_copy(eflat_hbm.at[pl.ds(E + e, CM)], didx.at[0], sem_d.at[0])
    pltpu.make_async_copy(eflat_hbm.at[pl.ds(0, CM)], sidx.at[0], sem_i.at[0]).wait()
    pltpu.async_copy(g_hbm.at[sidx.at[0]], rows.at[0], sem_g.at[0])
    pltpu.make_async_copy(g_hbm.at[sidx.at[0]], rows.at[0], sem_g.at[0]).wait()
    pltpu.make_async_copy(eflat_hbm.at[pl.ds(0, CM)], didx.at[0], sem_d.at[0]).wait()
    pltpu.async_copy(rows.at[0], acc_sh.at[didx.at[0]], sem_s.at[0], add=True)
    pltpu.make_async_copy(rows.at[0], acc_sh.at[didx.at[0]], sem_s.at[0]).wait()
    for b in range(1, NB_M):
        pltpu.make_async_copy(rows.at[b], acc_sh.at[didx.at[b]],
                              sem_s.at[b]).wait()
    plsc.subcore_barrier()
    pltpu.sync_copy(acc_sh.at[pl.ds(tid * RPT, RPT)],
                    out_hbm.at[core].at[pl.ds(tid * RPT, RPT)])


def _message_partials(eflat, g):
    return pl.kernel(
        _gather_body,
        out_type=jax.ShapeDtypeStruct((NC, NPAD, D), jnp.float32),
        mesh=_sc_mesh(),
        scratch_types=[
            pltpu.VMEM_SHARED((NPAD, D), jnp.float32),
            pltpu.VMEM((NB_M, CM), jnp.int32),
            pltpu.VMEM((NB_M, CM), jnp.int32),
            pltpu.VMEM((NB_M, CM, D), jnp.float32),
            pltpu.SemaphoreType.DMA((NB_M,)),
            pltpu.SemaphoreType.DMA((NB_M,)),
            pltpu.SemaphoreType.DMA((NB_M,)),
            pltpu.SemaphoreType.DMA((NB_M,)),
        ],
    )(eflat, g)


R = 2000


def _dis_from(dp_ref):
    deg = dp_ref[0, :, 0] + dp_ref[1, :, 0] + 1.0
    return lax.rsqrt(deg)


def _mm(a, w):
    return lax.dot_general(a, w, (((1,), (1,)), ((), ())),
                           preferred_element_type=jnp.float32)


def _tc_first_body(dp_ref, x_ref, w1_ref, g1_ref):
    dis = _dis_from(dp_ref)
    g1_ref[...] = _mm(x_ref[...], w1_ref[...]) * dis[:, None]


def _tc_mid_body(dp_ref, acc_ref, g1_ref, b1_ref, w2_ref, x1_ref, g2_ref):
    dis = _dis_from(dp_ref)
    t = (acc_ref[0] + acc_ref[1] + g1_ref[...]) * dis[:, None] + b1_ref[...]
    x1 = jnp.maximum(t, 0.0)
    x1_ref[...] = x1
    g2_ref[...] = _mm(x1, w2_ref[...]) * dis[:, None]


def _tc_last_body(dp_ref, acc_ref, g2_ref, b2_ref, x1_ref,
                  wp1a_ref, wp1b_ref, bp1_ref, wp2_ref, bp2_ref, out_ref):
    dis = _dis_from(dp_ref)
    t = (acc_ref[0] + acc_ref[1] + g2_ref[...]) * dis[:, None] + b2_ref[...]
    x2 = jnp.maximum(t, 0.0)
    p = _mm(x1_ref[...], wp1a_ref[...]) + _mm(x2, wp1b_ref[...]) + bp1_ref[...]
    p = jnp.maximum(p, 0.0)
    out_ref[...] = _mm(p, wp2_ref[...]) + bp2_ref[...]


def _row_spec(width):
    return pl.BlockSpec((R, width), lambda i: (i, 0))


_DP_SPEC = pl.BlockSpec((NC, R, DEG_W), lambda i: (0, i, 0))
_ACC_SPEC = pl.BlockSpec((NC, R, D), lambda i: (0, i, 0))


def _full_spec(r, c):
    return pl.BlockSpec((r, c), lambda i: (0, 0))


def _tc_first(dp, x, w1):
    return pl.pallas_call(
        _tc_first_body,
        grid=(N // R,),
        in_specs=[_DP_SPEC, _row_spec(D), _full_spec(D, D)],
        out_specs=_row_spec(D),
        out_shape=jax.ShapeDtypeStruct((N, D), jnp.float32),
    )(dp, x, w1)


def _tc_mid(dp, acc, g1, b1, w2):
    return pl.pallas_call(
        _tc_mid_body,
        grid=(N // R,),
        in_specs=[_DP_SPEC, _ACC_SPEC, _row_spec(D), _full_spec(1, D),
                  _full_spec(D, D)],
        out_specs=[_row_spec(D), _row_spec(D)],
        out_shape=[jax.ShapeDtypeStruct((N, D), jnp.float32),
                   jax.ShapeDtypeStruct((N, D), jnp.float32)],
    )(dp, acc, g1, b1, w2)


def _tc_last(dp, acc, g2, b2, x1, wp1a, wp1b, bp1, wp2, bp2):
    return pl.pallas_call(
        _tc_last_body,
        grid=(N // R,),
        in_specs=[_DP_SPEC, _ACC_SPEC, _row_spec(D), _full_spec(1, D),
                  _row_spec(D), _full_spec(D, D), _full_spec(D, D),
                  _full_spec(1, D), _full_spec(D, D), _full_spec(1, D)],
        out_specs=_row_spec(D),
        out_shape=jax.ShapeDtypeStruct((N, D), jnp.float32),
    )(dp, acc, g2, b2, x1, wp1a, wp1b, bp1, wp2, bp2)


def kernel(x, edge_index, W1, b1, W2, b2, Wp1, bp1, Wp2, bp2):
    eflat = edge_index.reshape(2 * E)

    dp = _degree_partials(eflat)

    g1 = _tc_first(dp, x, W1)
    acc1 = _message_partials(eflat, g1)
    x1, g2 = _tc_mid(dp, acc1, g1, b1.reshape(1, D), W2)
    acc2 = _message_partials(eflat, g2)
    out = _tc_last(dp, acc2, g2, b2.reshape(1, D), x1,
                   Wp1[:, :D], Wp1[:, D:], bp1.reshape(1, D), Wp2,
                   bp2.reshape(1, D))
    return out
